# Initial kernel scaffold; baseline (speedup 1.0000x reference)
import jax
import jax.numpy as jnp
from jax import lax
from jax.experimental import pallas as pl
from jax.experimental.pallas import tpu as pltpu
from jax.experimental.pallas import tpu_sc as plsc

N = 10000
E = 320000
D = 128
NC = 2
NS = 16
L = 16
NW = NC * NS
ET = E // NW
PT = 10240
EROWS = NW * PT // 128
TA = EROWS // NS
TB = EROWS // NW
NP = 10240
NT = NP // NS


def _sc_body(wa_hbm, wb_hbm, src_hbm, dst_hbm, feat_hbm,
             z_out, den_out, deg_out,
             wbufA, dbufA, src16, dst16, w16, e16, val16, rows, zf, s_loc,
             s_acc, den_acc, deg_acc, z_acc, sem):
    c = lax.axis_index("c")
    s = lax.axis_index("s")
    base_n = s * NT

    def _z1(i, carry):
        zf[pl.ds(i * L, L)] = jnp.zeros((L,), jnp.float32)
        return carry
    lax.fori_loop(0, NT // L, _z1, None)

    def _z2(i, carry):
        rows[i >> 3, pl.ds((i & 7) * L, L)] = jnp.zeros((L,), jnp.float32)
        return carry
    lax.fori_loop(0, 128 * 8, _z2, None)

    pltpu.sync_copy(zf, s_acc.at[pl.ds(base_n, NT)])
    pltpu.sync_copy(zf, den_acc.at[pl.ds(base_n, NT)])
    pltpu.sync_copy(zf, deg_acc.at[pl.ds(base_n, NT)])
    for q in range(NT // 128):
        pltpu.sync_copy(rows, z_acc.at[pl.ds(base_n + q * 128, 128)])
    plsc.subcore_barrier()

    def _phA(ch, carry):
        r0 = s * TA + ch * 16
        pltpu.sync_copy(wa_hbm.at[pl.ds(r0, 16)], wbufA)
        pltpu.sync_copy(dst_hbm.at[pl.ds(r0, 16)], dbufA)

        def _scat(j, carry2):
            pltpu.sync_copy(wbufA.at[j], s_acc.at[dbufA.at[j]], add=True)
            return carry2
        return lax.fori_loop(0, 16, _scat, carry)
    lax.fori_loop(0, TA // 16, _phA, None)
    plsc.subcore_barrier()

    pltpu.sync_copy(s_acc, s_loc)

    g = c * NS + s

    def _phBC(r, carry):
        r0 = g * TB + r * 16
        pltpu.sync_copy(src_hbm.at[pl.ds(r0, 16)], src16)
        pltpu.sync_copy(dst_hbm.at[pl.ds(r0, 16)], dst16)
        pltpu.sync_copy(wb_hbm.at[pl.ds(r0, 16)], w16)

        def _eb(t, carry2):
            j = t >> 3
            k = (t & 7) * L
            wv = w16[j, pl.ds(k, L)]
            dv = dst16[j, pl.ds(k, L)]
            sv = plsc.load_gather(s_loc, [dv])
            ev = jnp.exp(-(wv / jnp.maximum(sv, 1e-12)))
            msk = wv >= 0.0
            e16[j, pl.ds(k, L)] = jnp.where(msk, ev, 0.0)
            val16[j, pl.ds(k, L)] = jnp.where(msk, 1.0, 0.0)
            return carry2
        lax.fori_loop(0, 128, _eb, carry)

        def _rowloop(j, carry2):
            pltpu.async_copy(feat_hbm.at[src16.at[j]], rows, sem).wait()

            def _scale(i, carry3):
                es = e16[j, i]
                for k in range(8):
                    sl = pl.ds(k * L, L)
                    rows[i, sl] = rows[i, sl] * es
                return carry3
            lax.fori_loop(0, 128, _scale, carry2)
            pltpu.sync_copy(rows, z_acc.at[dst16.at[j]], add=True)
            pltpu.sync_copy(e16.at[j], den_acc.at[dst16.at[j]], add=True)
            pltpu.sync_copy(val16.at[j], deg_acc.at[dst16.at[j]], add=True)
            return carry2
        return lax.fori_loop(0, 16, _rowloop, carry)
    lax.fori_loop(0, TB // 16, _phBC, None)
    plsc.subcore_barrier()

    pltpu.sync_copy(z_acc.at[pl.ds(base_n, NT)], z_out.at[c, pl.ds(base_n, NT)])
    pltpu.sync_copy(den_acc.at[pl.ds(base_n, NT)],
                    den_out.at[c, pl.ds(base_n, NT)])
    pltpu.sync_copy(deg_acc.at[pl.ds(base_n, NT)],
                    deg_out.at[c, pl.ds(base_n, NT)])


_sc_call = pl.kernel(
    _sc_body,
    out_type=(
        jax.ShapeDtypeStruct((NC, NP, D), jnp.float32),
        jax.ShapeDtypeStruct((NC, NP), jnp.float32),
        jax.ShapeDtypeStruct((NC, NP), jnp.float32),
    ),
    mesh=plsc.VectorSubcoreMesh(core_axis_name="c", subcore_axis_name="s",
                                num_cores=NC, num_subcores=NS),
    scratch_types=[
        pltpu.VMEM((16, 128), jnp.float32),
        pltpu.VMEM((16, 128), jnp.int32),
        pltpu.VMEM((16, 128), jnp.int32),
        pltpu.VMEM((16, 128), jnp.int32),
        pltpu.VMEM((16, 128), jnp.float32),
        pltpu.VMEM((16, 128), jnp.float32),
        pltpu.VMEM((16, 128), jnp.float32),
        pltpu.VMEM((128, D), jnp.float32),
        pltpu.VMEM((NT,), jnp.float32),
        pltpu.VMEM((NP,), jnp.float32),
        pltpu.VMEM_SHARED((NP,), jnp.float32),
        pltpu.VMEM_SHARED((NP,), jnp.float32),
        pltpu.VMEM_SHARED((NP,), jnp.float32),
        pltpu.VMEM_SHARED((NP, D), jnp.float32),
        pltpu.SemaphoreType.DMA,
    ],
)


def _tc_body(feat_ref, z0_ref, z1_ref, dd0_ref, dd1_ref,
             wp_ref, bp_ref, ws_ref, bs_ref, out_ref):
    z = z0_ref[...] + z1_ref[...]
    den = dd0_ref[:, 0:1] + dd1_ref[:, 0:1]
    deg = dd0_ref[:, 1:2] + dd1_ref[:, 1:2]
    rden = 1.0 / jnp.maximum(den, 1e-12)
    sfrac = den * rden
    rdeg = 1.0 / jnp.maximum(deg, 1.0)
    zn = z * rden
    m = jnp.dot(zn, wp_ref[...], preferred_element_type=jnp.float32)
    agg = (m + sfrac * bp_ref[...]) * rdeg
    out_ref[...] = (jnp.dot(feat_ref[...], ws_ref[...],
                            preferred_element_type=jnp.float32)
                    + bs_ref[...] + agg)


_BN = 500


def kernel(feat, edge_index, efeat, W_pool, b_pool, W_self, b_self):
    src = edge_index[0]
    dst = edge_index[1]
    w = efeat[:, 0]

    pad = PT - ET
    src2 = jnp.pad(src.reshape(NW, ET), ((0, 0), (0, pad))).reshape(EROWS, 128)
    dst2 = jnp.pad(dst.reshape(NW, ET), ((0, 0), (0, pad))).reshape(EROWS, 128)
    w2r = w.reshape(NW, ET)
    wa2 = jnp.pad(w2r, ((0, 0), (0, pad))).reshape(EROWS, 128)
    wb2 = jnp.pad(w2r, ((0, 0), (0, pad)),
                  constant_values=-1.0).reshape(EROWS, 128)

    z_part, den_part, deg_part = _sc_call(wa2, wb2, src2, dst2, feat)

    dd0 = jnp.stack([den_part[0, :N], deg_part[0, :N]], axis=-1)
    dd1 = jnp.stack([den_part[1, :N], deg_part[1, :N]], axis=-1)
    z0 = z_part[0, :N]
    z1 = z_part[1, :N]
    bp = b_pool.reshape(1, D)
    bs = b_self.reshape(1, D)

    out = pl.pallas_call(
        _tc_body,
        out_shape=jax.ShapeDtypeStruct((N, D), jnp.float32),
        grid=(N // _BN,),
        in_specs=[
            pl.BlockSpec((_BN, D), lambda i: (i, 0)),
            pl.BlockSpec((_BN, D), lambda i: (i, 0)),
            pl.BlockSpec((_BN, D), lambda i: (i, 0)),
            pl.BlockSpec((_BN, 2), lambda i: (i, 0)),
            pl.BlockSpec((_BN, 2), lambda i: (i, 0)),
            pl.BlockSpec((D, D), lambda i: (0, 0)),
            pl.BlockSpec((1, D), lambda i: (0, 0)),
            pl.BlockSpec((D, D), lambda i: (0, 0)),
            pl.BlockSpec((1, D), lambda i: (0, 0)),
        ],
        out_specs=pl.BlockSpec((_BN, D), lambda i: (i, 0)),
    )(feat, z0, z1, dd0, dd1, W_pool, bp, W_self, bs)
    return out


# speedup vs baseline: 14.2441x; 14.2441x over previous
import jax
import jax.numpy as jnp
from jax import lax
from jax.experimental import pallas as pl
from jax.experimental.pallas import tpu as pltpu
from jax.experimental.pallas import tpu_sc as plsc

N = 10000
E = 320000
D = 128
NC = 2
NS = 16
L = 16
NW = NC * NS
ET = E // NW
PT = 10240
EROWS = NW * PT // 128
TA = EROWS // NS
TB = EROWS // NW
NP = 10240
NT = NP // NS


def _sc_body(wa_hbm, wb_hbm, src_hbm, dst_hbm, feat_hbm,
             z_out, den_out, deg_out,
             wbufA, dbufA, src16, dst16, w16, e16, val16, rows, zf, s_loc,
             s_acc, den_acc, deg_acc, z_acc, sem):
    c = lax.axis_index("c")
    s = lax.axis_index("s")
    base_n = s * NT

    def _z1(i, carry):
        zf[pl.ds(i * L, L)] = jnp.zeros((L,), jnp.float32)
        return carry
    lax.fori_loop(0, NT // L, _z1, None)

    def _z2(i, carry):
        rows[i >> 3, pl.ds((i & 7) * L, L)] = jnp.zeros((L,), jnp.float32)
        return carry
    lax.fori_loop(0, 128 * 8, _z2, None)

    pltpu.sync_copy(zf, s_acc.at[pl.ds(base_n, NT)])
    pltpu.sync_copy(zf, den_acc.at[pl.ds(base_n, NT)])
    pltpu.sync_copy(zf, deg_acc.at[pl.ds(base_n, NT)])
    for q in range(NT // 128):
        pltpu.sync_copy(rows, z_acc.at[pl.ds(base_n + q * 128, 128)])
    plsc.subcore_barrier()

    def _phA(ch, carry):
        r0 = s * TA + ch * 16
        pltpu.sync_copy(wa_hbm.at[pl.ds(r0, 16)], wbufA)
        pltpu.sync_copy(dst_hbm.at[pl.ds(r0, 16)], dbufA)

        def _scat(j, carry2):
            pltpu.sync_copy(wbufA.at[j], s_acc.at[dbufA.at[j]], add=True)
            return carry2
        return lax.fori_loop(0, 16, _scat, carry)
    lax.fori_loop(0, TA // 16, _phA, None)
    plsc.subcore_barrier()

    pltpu.sync_copy(s_acc, s_loc)

    g = c * NS + s

    def _phBC(r, carry):
        r0 = g * TB + r * 16
        pltpu.sync_copy(src_hbm.at[pl.ds(r0, 16)], src16)
        pltpu.sync_copy(dst_hbm.at[pl.ds(r0, 16)], dst16)
        pltpu.sync_copy(wb_hbm.at[pl.ds(r0, 16)], w16)

        def _eb(t, carry2):
            j = t >> 3
            k = (t & 7) * L
            wv = w16[j, pl.ds(k, L)]
            dv = dst16[j, pl.ds(k, L)]
            sv = plsc.load_gather(s_loc, [dv])
            ev = jnp.exp(-(wv / jnp.maximum(sv, 1e-12)))
            msk = wv >= 0.0
            e16[j, pl.ds(k, L)] = jnp.where(msk, ev, 0.0)
            val16[j, pl.ds(k, L)] = jnp.where(msk, 1.0, 0.0)
            return carry2
        lax.fori_loop(0, 128, _eb, carry)

        def _rowloop(j, carry2):
            pltpu.async_copy(feat_hbm.at[src16.at[j]], rows, sem).wait()

            def _scale(i, carry3):
                jv = jnp.full((L,), j, jnp.int32)
                iv = jnp.full((L,), i, jnp.int32)
                es = plsc.load_gather(e16, [jv, iv])
                for k in range(8):
                    sl = pl.ds(k * L, L)
                    rows[i, sl] = rows[i, sl] * es
                return carry3
            lax.fori_loop(0, 128, _scale, carry2)
            pltpu.sync_copy(rows, z_acc.at[dst16.at[j]], add=True)
            pltpu.sync_copy(e16.at[j], den_acc.at[dst16.at[j]], add=True)
            pltpu.sync_copy(val16.at[j], deg_acc.at[dst16.at[j]], add=True)
            return carry2
        return lax.fori_loop(0, 16, _rowloop, carry)
    lax.fori_loop(0, TB // 16, _phBC, None)
    plsc.subcore_barrier()

    pltpu.sync_copy(z_acc.at[pl.ds(base_n, NT)], z_out.at[c, pl.ds(base_n, NT)])
    pltpu.sync_copy(den_acc.at[pl.ds(base_n, NT)],
                    den_out.at[c, pl.ds(base_n, NT)])
    pltpu.sync_copy(deg_acc.at[pl.ds(base_n, NT)],
                    deg_out.at[c, pl.ds(base_n, NT)])


_sc_call = pl.kernel(
    _sc_body,
    out_type=(
        jax.ShapeDtypeStruct((NC, NP, D), jnp.float32),
        jax.ShapeDtypeStruct((NC, NP), jnp.float32),
        jax.ShapeDtypeStruct((NC, NP), jnp.float32),
    ),
    mesh=plsc.VectorSubcoreMesh(core_axis_name="c", subcore_axis_name="s",
                                num_cores=NC, num_subcores=NS),
    compiler_params=pltpu.CompilerParams(needs_layout_passes=False),
    scratch_types=[
        pltpu.VMEM((16, 128), jnp.float32),
        pltpu.VMEM((16, 128), jnp.int32),
        pltpu.VMEM((16, 128), jnp.int32),
        pltpu.VMEM((16, 128), jnp.int32),
        pltpu.VMEM((16, 128), jnp.float32),
        pltpu.VMEM((16, 128), jnp.float32),
        pltpu.VMEM((16, 128), jnp.float32),
        pltpu.VMEM((128, D), jnp.float32),
        pltpu.VMEM((NT,), jnp.float32),
        pltpu.VMEM((NP,), jnp.float32),
        pltpu.VMEM_SHARED((NP,), jnp.float32),
        pltpu.VMEM_SHARED((NP,), jnp.float32),
        pltpu.VMEM_SHARED((NP,), jnp.float32),
        pltpu.VMEM_SHARED((NP, D), jnp.float32),
        pltpu.SemaphoreType.DMA,
    ],
)


def _tc_body(feat_ref, z0_ref, z1_ref, dd0_ref, dd1_ref,
             wp_ref, bp_ref, ws_ref, bs_ref, out_ref):
    z = z0_ref[...] + z1_ref[...]
    den = dd0_ref[:, 0:1] + dd1_ref[:, 0:1]
    deg = dd0_ref[:, 1:2] + dd1_ref[:, 1:2]
    rden = 1.0 / jnp.maximum(den, 1e-12)
    sfrac = den * rden
    rdeg = 1.0 / jnp.maximum(deg, 1.0)
    zn = z * rden
    m = jnp.dot(zn, wp_ref[...], preferred_element_type=jnp.float32)
    agg = (m + sfrac * bp_ref[...]) * rdeg
    out_ref[...] = (jnp.dot(feat_ref[...], ws_ref[...],
                            preferred_element_type=jnp.float32)
                    + bs_ref[...] + agg)


_BN = 1000


def kernel(feat, edge_index, efeat, W_pool, b_pool, W_self, b_self):
    src = edge_index[0]
    dst = edge_index[1]
    w = efeat[:, 0]

    pad = PT - ET
    src2 = jnp.pad(src.reshape(NW, ET), ((0, 0), (0, pad))).reshape(EROWS, 128)
    dst2 = jnp.pad(dst.reshape(NW, ET), ((0, 0), (0, pad))).reshape(EROWS, 128)
    w2r = w.reshape(NW, ET)
    wa2 = jnp.pad(w2r, ((0, 0), (0, pad))).reshape(EROWS, 128)
    wb2 = jnp.pad(w2r, ((0, 0), (0, pad)),
                  constant_values=-1.0).reshape(EROWS, 128)

    z_part, den_part, deg_part = _sc_call(wa2, wb2, src2, dst2, feat)

    dd0 = jnp.stack([den_part[0, :N], deg_part[0, :N]], axis=-1)
    dd1 = jnp.stack([den_part[1, :N], deg_part[1, :N]], axis=-1)
    z0 = z_part[0, :N]
    z1 = z_part[1, :N]
    bp = b_pool.reshape(1, D)
    bs = b_self.reshape(1, D)

    out = pl.pallas_call(
        _tc_body,
        out_shape=jax.ShapeDtypeStruct((N, D), jnp.float32),
        grid=(N // _BN,),
        in_specs=[
            pl.BlockSpec((_BN, D), lambda i: (i, 0)),
            pl.BlockSpec((_BN, D), lambda i: (i, 0)),
            pl.BlockSpec((_BN, D), lambda i: (i, 0)),
            pl.BlockSpec((_BN, 2), lambda i: (i, 0)),
            pl.BlockSpec((_BN, 2), lambda i: (i, 0)),
            pl.BlockSpec((D, D), lambda i: (0, 0)),
            pl.BlockSpec((1, D), lambda i: (0, 0)),
            pl.BlockSpec((D, D), lambda i: (0, 0)),
            pl.BlockSpec((1, D), lambda i: (0, 0)),
        ],
        out_specs=pl.BlockSpec((_BN, D), lambda i: (i, 0)),
    )(feat, z0, z1, dd0, dd1, W_pool, bp, W_self, bs)
    return out
